# Initial kernel scaffold; baseline (speedup 1.0000x reference)
import functools

import jax
import jax.numpy as jnp
from jax import lax
from jax.experimental import pallas as pl
from jax.experimental.pallas import tpu as pltpu
from jax.experimental.pallas import tpu_sc as plsc

N = 10000
E = 320000
D = 128
ALPHA = 0.2

NC = 2
NS = 16
NT = NC * NS
L = 16
EPT = E // NT
B = 80
NB = EPT // B
RPT = N // NS

_mesh = plsc.VectorSubcoreMesh(core_axis_name="c", subcore_axis_name="s")


def _mm_body(x_ref, w_ref, a01_ref, xp_ref, s_ref):
    xp = jnp.dot(x_ref[...], w_ref[...], preferred_element_type=jnp.float32)
    xp_ref[...] = xp
    s_ref[...] = jnp.dot(xp, a01_ref[...], preferred_element_type=jnp.float32)


def _matmul(X, W, a01):
    return pl.pallas_call(
        _mm_body,
        out_shape=[
            jax.ShapeDtypeStruct((N, D), jnp.float32),
            jax.ShapeDtypeStruct((N, 2), jnp.float32),
        ],
    )(X, W, a01)


@functools.partial(
    pl.kernel,
    out_type=[
        jax.ShapeDtypeStruct((E,), jnp.float32),
        jax.ShapeDtypeStruct((NT, L), jnp.float32),
        jax.ShapeDtypeStruct((NT, L), jnp.float32),
    ],
    mesh=_mesh,
    scratch_types=[
        pltpu.VMEM((N, 2), jnp.float32),
        pltpu.VMEM((EPT,), jnp.int32),
        pltpu.VMEM((EPT,), jnp.int32),
        pltpu.VMEM((EPT,), jnp.float32),
        pltpu.VMEM((L,), jnp.float32),
        pltpu.VMEM((L,), jnp.float32),
    ],
)
def _att_kernel(s01_hbm, row_hbm, col_hbm, att_hbm, mn_hbm, mx_hbm,
                s01_v, row_v, col_v, att_v, mn_v, mx_v):
    c = lax.axis_index("c")
    s = lax.axis_index("s")
    wid = c * NS + s
    base = wid * EPT
    pltpu.sync_copy(s01_hbm, s01_v)
    pltpu.sync_copy(row_hbm.at[pl.ds(base, EPT)], row_v)
    pltpu.sync_copy(col_hbm.at[pl.ds(base, EPT)], col_v)
    zero16 = jnp.zeros((L,), jnp.int32)
    one16 = jnp.ones((L,), jnp.int32)

    def body(i, carry):
        mn, mx = carry
        off = i * L
        ridx = row_v[pl.ds(off, L)]
        cidx = col_v[pl.ds(off, L)]
        v0 = plsc.load_gather(s01_v, [ridx, zero16])
        v1 = plsc.load_gather(s01_v, [cidx, one16])
        att = v0 + v1
        att = jnp.where(att > 0, att, ALPHA * att)
        att_v[pl.ds(off, L)] = att
        return jnp.minimum(mn, att), jnp.maximum(mx, att)

    big = jnp.full((L,), jnp.inf, dtype=jnp.float32)
    mn, mx = lax.fori_loop(0, EPT // L, body, (big, -big))
    mn_v[...] = mn
    mx_v[...] = mx
    pltpu.sync_copy(att_v, att_hbm.at[pl.ds(base, EPT)])
    pltpu.sync_copy(mn_v, mn_hbm.at[wid])
    pltpu.sync_copy(mx_v, mx_hbm.at[wid])


@functools.partial(
    pl.kernel,
    out_type=[
        jax.ShapeDtypeStruct((NC, N, D), jnp.float32),
        jax.ShapeDtypeStruct((NC, N), jnp.float32),
    ],
    mesh=_mesh,
    scratch_types=[
        pltpu.VMEM_SHARED((N, D), jnp.float32),
        pltpu.VMEM_SHARED((N,), jnp.float32),
        pltpu.VMEM((EPT,), jnp.float32),
        pltpu.VMEM((NT, L), jnp.float32),
        pltpu.VMEM((NT, L), jnp.float32),
        pltpu.VMEM((B,), jnp.int32),
        pltpu.VMEM((B,), jnp.int32),
        pltpu.VMEM((B, D), jnp.float32),
        pltpu.SemaphoreType.DMA,
    ],
)
def _spmm_kernel(att_hbm, mn_hbm, mx_hbm, row_hbm, col_hbm, xp_hbm,
                 h_out, rs_out,
                 acc, rs_acc, p_v, mn_v, mx_v, ridx, cidx, gbuf, sem):
    c = lax.axis_index("c")
    s = lax.axis_index("s")
    wid = c * NS + s
    base = wid * EPT

    zvec = jnp.zeros((L,), jnp.float32)

    def zb(i, _):
        for k in range(D // L):
            gbuf[i, pl.ds(k * L, L)] = zvec
        return 0

    lax.fori_loop(0, B, zb, 0)
    for j in range(5):
        pltpu.sync_copy(gbuf.at[pl.ds(0, 125)],
                        acc.at[pl.ds(s * RPT + j * 125, 125)])

    def zp(i, _):
        p_v[pl.ds(i * L, L)] = zvec
        return 0

    lax.fori_loop(0, 1000 // L, zp, 0)

    @pl.when(s < 10)
    def _():
        pltpu.sync_copy(p_v.at[pl.ds(0, 1000)],
                        rs_acc.at[pl.ds(s * 1000, 1000)])

    pltpu.sync_copy(mn_hbm, mn_v)
    pltpu.sync_copy(mx_hbm, mx_v)
    mn = mn_v[0]
    mx = mx_v[0]
    for i in range(1, NT):
        mn = jnp.minimum(mn, mn_v[i])
        mx = jnp.maximum(mx, mx_v[i])
    mn_s = jnp.min(mn)
    mx_s = jnp.max(mx)
    inv = 1.0 / (mx_s - mn_s)

    pltpu.sync_copy(att_hbm.at[pl.ds(base, EPT)], p_v)

    def pb(i, _):
        off = i * L
        v = p_v[pl.ds(off, L)]
        p_v[pl.ds(off, L)] = jnp.exp((v - mn_s) * inv)
        return 0

    lax.fori_loop(0, EPT // L, pb, 0)

    plsc.subcore_barrier()

    def batch(j, _):
        eoff = base + j * B
        pltpu.sync_copy(row_hbm.at[pl.ds(eoff, B)], ridx)
        pltpu.sync_copy(col_hbm.at[pl.ds(eoff, B)], cidx)
        pltpu.async_copy(xp_hbm.at[cidx], gbuf, sem).wait()

        def sb(i, _):
            a = p_v[j * B + i]
            for k in range(D // L):
                gbuf[i, pl.ds(k * L, L)] = gbuf[i, pl.ds(k * L, L)] * a
            return 0

        lax.fori_loop(0, B, sb, 0)
        pltpu.sync_copy(gbuf, acc.at[ridx], add=True)
        pltpu.sync_copy(p_v.at[pl.ds(j * B, B)], rs_acc.at[ridx], add=True)
        return 0

    lax.fori_loop(0, NB, batch, 0)

    plsc.subcore_barrier()

    pltpu.sync_copy(acc.at[pl.ds(s * RPT, RPT)],
                    h_out.at[c, pl.ds(s * RPT, RPT)])

    @pl.when(s < 10)
    def _():
        pltpu.sync_copy(rs_acc.at[pl.ds(s * 1000, 1000)],
                        rs_out.at[c, pl.ds(s * 1000, 1000)])


def _epi_body(hp_ref, rs_ref, out_ref):
    h = hp_ref[0] + hp_ref[1]
    rs = rs_ref[0] + rs_ref[1]
    out_ref[...] = h / rs


def _epilogue(h_part, rs_part):
    return pl.pallas_call(
        _epi_body,
        out_shape=jax.ShapeDtypeStruct((N, D), jnp.float32),
    )(h_part, rs_part)


def kernel(X, edge_index, W, a0, a1):
    row = edge_index[0].astype(jnp.int32)
    col = edge_index[1].astype(jnp.int32)
    a01 = jnp.concatenate([a0.T, a1.T], axis=1)
    xp, s01 = _matmul(X, W, a01)
    att, mns, mxs = _att_kernel(s01, row, col)
    h_part, rs_part = _spmm_kernel(att, mns, mxs, row, col, xp)
    return _epilogue(h_part, rs_part.reshape(NC, N, 1))


# speedup vs baseline: 14.8514x; 14.8514x over previous
import functools

import jax
import jax.numpy as jnp
from jax import lax
from jax.experimental import pallas as pl
from jax.experimental.pallas import tpu as pltpu
from jax.experimental.pallas import tpu_sc as plsc

N = 10000
E = 320000
D = 128
ALPHA = 0.2

NC = 2
NS = 16
NT = NC * NS
L = 16
EPT = E // NT
B = 80
NB = EPT // B
NP = 10240
RPT = NP // NS

_mesh = plsc.VectorSubcoreMesh(core_axis_name="c", subcore_axis_name="s")


def _mm_body(x_ref, w_ref, a01_ref, xp_ref, s_ref):
    xp = jnp.dot(x_ref[...], w_ref[...], preferred_element_type=jnp.float32)
    xp_ref[...] = xp
    s_ref[...] = jnp.dot(xp, a01_ref[...], preferred_element_type=jnp.float32)


def _matmul(X, W, a01):
    return pl.pallas_call(
        _mm_body,
        out_shape=[
            jax.ShapeDtypeStruct((N, D), jnp.float32),
            jax.ShapeDtypeStruct((N, 2), jnp.float32),
        ],
    )(X, W, a01)


@functools.partial(
    pl.kernel,
    out_type=[
        jax.ShapeDtypeStruct((E,), jnp.float32),
        jax.ShapeDtypeStruct((NT * L,), jnp.float32),
        jax.ShapeDtypeStruct((NT * L,), jnp.float32),
    ],
    mesh=_mesh,
    compiler_params=pltpu.CompilerParams(needs_layout_passes=False),
    scratch_types=[
        pltpu.VMEM((2 * N,), jnp.float32),
        pltpu.VMEM((EPT,), jnp.int32),
        pltpu.VMEM((EPT,), jnp.int32),
        pltpu.VMEM((EPT,), jnp.float32),
        pltpu.VMEM((L,), jnp.float32),
        pltpu.VMEM((L,), jnp.float32),
    ],
)
def _att_kernel(s01_hbm, row_hbm, col_hbm, att_hbm, mn_hbm, mx_hbm,
                s01_v, row_v, col_v, att_v, mn_v, mx_v):
    c = lax.axis_index("c")
    s = lax.axis_index("s")
    wid = c * NS + s
    base = wid * EPT
    pltpu.sync_copy(s01_hbm, s01_v)
    pltpu.sync_copy(row_hbm.at[pl.ds(base, EPT)], row_v)
    pltpu.sync_copy(col_hbm.at[pl.ds(base, EPT)], col_v)
    def body(i, carry):
        mn, mx = carry
        off = i * L
        ridx = row_v[pl.ds(off, L)]
        cidx = col_v[pl.ds(off, L)]
        v0 = plsc.load_gather(s01_v, [2 * ridx])
        v1 = plsc.load_gather(s01_v, [2 * cidx + 1])
        att = v0 + v1
        att = jnp.where(att > 0, att, ALPHA * att)
        att_v[pl.ds(off, L)] = att
        return jnp.minimum(mn, att), jnp.maximum(mx, att)

    big = jnp.full((L,), jnp.inf, dtype=jnp.float32)
    mn, mx = lax.fori_loop(0, EPT // L, body, (big, -big))
    mn_v[...] = mn
    mx_v[...] = mx
    pltpu.sync_copy(att_v, att_hbm.at[pl.ds(base, EPT)])
    pltpu.sync_copy(mn_v, mn_hbm.at[pl.ds(wid * L, L)])
    pltpu.sync_copy(mx_v, mx_hbm.at[pl.ds(wid * L, L)])


@functools.partial(
    pl.kernel,
    out_type=[
        jax.ShapeDtypeStruct((NC, NP, D), jnp.float32),
        jax.ShapeDtypeStruct((NC, NP), jnp.float32),
    ],
    mesh=_mesh,
    compiler_params=pltpu.CompilerParams(needs_layout_passes=False),
    scratch_types=[
        pltpu.VMEM_SHARED((NP, D), jnp.float32),
        pltpu.VMEM_SHARED((NP,), jnp.float32),
        pltpu.VMEM((EPT,), jnp.float32),
        pltpu.VMEM((NT * L,), jnp.float32),
        pltpu.VMEM((NT * L,), jnp.float32),
        pltpu.VMEM((B,), jnp.int32),
        pltpu.VMEM((B,), jnp.int32),
        pltpu.VMEM((B, D), jnp.float32),
        pltpu.SemaphoreType.DMA,
    ],
)
def _spmm_kernel(att_hbm, mn_hbm, mx_hbm, row_hbm, col_hbm, xp_hbm,
                 h_out, rs_out,
                 acc, rs_acc, p_v, mn_v, mx_v, ridx, cidx, gbuf, sem):
    c = lax.axis_index("c")
    s = lax.axis_index("s")
    wid = c * NS + s
    base = wid * EPT

    zvec = jnp.zeros((L,), jnp.float32)

    def zb(i, _):
        for k in range(D // L):
            gbuf[i, pl.ds(k * L, L)] = zvec
        return 0

    lax.fori_loop(0, B, zb, 0)
    for j in range(RPT // B):
        pltpu.sync_copy(gbuf, acc.at[pl.ds(s * RPT + j * B, B)])

    def zp(i, _):
        p_v[pl.ds(i * L, L)] = zvec
        return 0

    lax.fori_loop(0, RPT // L, zp, 0)
    pltpu.sync_copy(p_v.at[pl.ds(0, RPT)], rs_acc.at[pl.ds(s * RPT, RPT)])

    pltpu.sync_copy(mn_hbm, mn_v)
    pltpu.sync_copy(mx_hbm, mx_v)
    mn = mn_v[pl.ds(0, L)]
    mx = mx_v[pl.ds(0, L)]
    for i in range(1, NT):
        mn = jnp.minimum(mn, mn_v[pl.ds(i * L, L)])
        mx = jnp.maximum(mx, mx_v[pl.ds(i * L, L)])
    mn_s = jnp.min(mn)
    mx_s = jnp.max(mx)
    inv_v = 1.0 / jnp.full((L,), mx_s - mn_s, dtype=jnp.float32)

    pltpu.sync_copy(att_hbm.at[pl.ds(base, EPT)], p_v)

    def pb(i, _):
        off = i * L
        v = p_v[pl.ds(off, L)]
        p_v[pl.ds(off, L)] = jnp.exp((v - mn_s) * inv_v)
        return 0

    lax.fori_loop(0, EPT // L, pb, 0)

    plsc.subcore_barrier()

    def batch(j, _):
        eoff = base + j * B
        pltpu.sync_copy(row_hbm.at[pl.ds(eoff, B)], ridx)
        pltpu.sync_copy(col_hbm.at[pl.ds(eoff, B)], cidx)
        pltpu.async_copy(xp_hbm.at[cidx], gbuf, sem).wait()

        def sb(g, _):
            pv = p_v[pl.ds(j * B + g * L, L)]
            for l in range(L):
                a = pv[l]
                r = g * L + l
                for k in range(D // L):
                    gbuf[r, pl.ds(k * L, L)] = gbuf[r, pl.ds(k * L, L)] * a
            return 0

        lax.fori_loop(0, B // L, sb, 0)
        pltpu.sync_copy(gbuf, acc.at[ridx], add=True)
        pltpu.sync_copy(p_v.at[pl.ds(j * B, B)], rs_acc.at[ridx], add=True)
        return 0

    lax.fori_loop(0, NB, batch, 0)

    plsc.subcore_barrier()

    pltpu.sync_copy(acc.at[pl.ds(s * RPT, RPT)],
                    h_out.at[c, pl.ds(s * RPT, RPT)])
    pltpu.sync_copy(rs_acc.at[pl.ds(s * RPT, RPT)],
                    rs_out.at[c, pl.ds(s * RPT, RPT)])


def _epi_body(hp_ref, rs_ref, out_ref):
    h = hp_ref[0, :N] + hp_ref[1, :N]
    rs = rs_ref[0, :N] + rs_ref[1, :N]
    out_ref[...] = h / rs


def _epilogue(h_part, rs_part):
    return pl.pallas_call(
        _epi_body,
        out_shape=jax.ShapeDtypeStruct((N, D), jnp.float32),
    )(h_part, rs_part)


def kernel(X, edge_index, W, a0, a1):
    row = edge_index[0].astype(jnp.int32)
    col = edge_index[1].astype(jnp.int32)
    a01 = jnp.concatenate([a0.T, a1.T], axis=1)
    xp, s01 = _matmul(X, W, a01)
    att, mns, mxs = _att_kernel(s01.reshape(2 * N), row, col)
    h_part, rs_part = _spmm_kernel(att, mns, mxs, row, col, xp)
    return _epilogue(h_part, rs_part.reshape(NC, NP, 1))


# speedup vs baseline: 30.0525x; 2.0235x over previous
import functools

import jax
import jax.numpy as jnp
from jax import lax
from jax.experimental import pallas as pl
from jax.experimental.pallas import tpu as pltpu
from jax.experimental.pallas import tpu_sc as plsc

N = 10000
E = 320000
D = 128
ALPHA = 0.2

NC = 2
NS = 16
NT = NC * NS
L = 16
EPT = E // NT
B = 80
NB = EPT // B
GB = 3
IR = 6
NP = 10240
RPT = NP // NS

_mesh = plsc.VectorSubcoreMesh(core_axis_name="c", subcore_axis_name="s")


def _mm_body(x_ref, w_ref, a01_ref, xp_ref, s_ref):
    xp = jnp.dot(x_ref[...], w_ref[...], preferred_element_type=jnp.float32)
    xp_ref[...] = xp
    s_ref[...] = jnp.dot(xp, a01_ref[...], preferred_element_type=jnp.float32)


def _matmul(X, W, a01):
    return pl.pallas_call(
        _mm_body,
        out_shape=[
            jax.ShapeDtypeStruct((N, D), jnp.float32),
            jax.ShapeDtypeStruct((N, 2), jnp.float32),
        ],
    )(X, W, a01)


@functools.partial(
    pl.kernel,
    out_type=[
        jax.ShapeDtypeStruct((E,), jnp.float32),
        jax.ShapeDtypeStruct((NT * L,), jnp.float32),
        jax.ShapeDtypeStruct((NT * L,), jnp.float32),
    ],
    mesh=_mesh,
    compiler_params=pltpu.CompilerParams(needs_layout_passes=False),
    scratch_types=[
        pltpu.VMEM((2 * N,), jnp.float32),
        pltpu.VMEM((EPT,), jnp.int32),
        pltpu.VMEM((EPT,), jnp.int32),
        pltpu.VMEM((EPT,), jnp.float32),
        pltpu.VMEM((L,), jnp.float32),
        pltpu.VMEM((L,), jnp.float32),
    ],
)
def _att_kernel(s01_hbm, row_hbm, col_hbm, att_hbm, mn_hbm, mx_hbm,
                s01_v, row_v, col_v, att_v, mn_v, mx_v):
    c = lax.axis_index("c")
    s = lax.axis_index("s")
    wid = c * NS + s
    base = wid * EPT
    pltpu.sync_copy(s01_hbm, s01_v)
    pltpu.sync_copy(row_hbm.at[pl.ds(base, EPT)], row_v)
    pltpu.sync_copy(col_hbm.at[pl.ds(base, EPT)], col_v)
    def body(i, carry):
        mn, mx = carry
        off = i * L
        ridx = row_v[pl.ds(off, L)]
        cidx = col_v[pl.ds(off, L)]
        v0 = plsc.load_gather(s01_v, [2 * ridx])
        v1 = plsc.load_gather(s01_v, [2 * cidx + 1])
        att = v0 + v1
        att = jnp.where(att > 0, att, ALPHA * att)
        att_v[pl.ds(off, L)] = att
        return jnp.minimum(mn, att), jnp.maximum(mx, att)

    big = jnp.full((L,), jnp.inf, dtype=jnp.float32)
    mn, mx = lax.fori_loop(0, EPT // L, body, (big, -big))
    mn_v[...] = mn
    mx_v[...] = mx
    pltpu.sync_copy(att_v, att_hbm.at[pl.ds(base, EPT)])
    pltpu.sync_copy(mn_v, mn_hbm.at[pl.ds(wid * L, L)])
    pltpu.sync_copy(mx_v, mx_hbm.at[pl.ds(wid * L, L)])


@functools.partial(
    pl.kernel,
    out_type=[
        jax.ShapeDtypeStruct((NC, NP, D), jnp.float32),
        jax.ShapeDtypeStruct((NC, NP), jnp.float32),
    ],
    mesh=_mesh,
    compiler_params=pltpu.CompilerParams(needs_layout_passes=False),
    scratch_types=[
        pltpu.VMEM_SHARED((NP, D), jnp.float32),
        pltpu.VMEM_SHARED((NP,), jnp.float32),
        pltpu.VMEM((EPT,), jnp.float32),
        pltpu.VMEM((NT * L,), jnp.float32),
        pltpu.VMEM((NT * L,), jnp.float32),
        pltpu.VMEM((IR, B), jnp.int32),
        pltpu.VMEM((IR, B), jnp.int32),
        pltpu.VMEM((B, D), jnp.float32),
        pltpu.VMEM((B, D), jnp.float32),
        pltpu.VMEM((B, D), jnp.float32),
        pltpu.SemaphoreType.DMA((GB,)),
        pltpu.SemaphoreType.DMA((GB,)),
        pltpu.SemaphoreType.DMA((GB,)),
        pltpu.SemaphoreType.DMA((IR,)),
    ],
)
def _spmm_kernel(att_hbm, mn_hbm, mx_hbm, row_hbm, col_hbm, xp_hbm,
                 h_out, rs_out,
                 acc, rs_acc, p_v, mn_v, mx_v, ridx, cidx,
                 gbuf0, gbuf1, gbuf2,
                 gsem, ssem, rsem, isem):
    gbufs = (gbuf0, gbuf1, gbuf2)
    c = lax.axis_index("c")
    s = lax.axis_index("s")
    wid = c * NS + s
    base = wid * EPT

    zvec = jnp.zeros((L,), jnp.float32)

    def zb(i, _):
        for k in range(D // L):
            gbuf0[i, pl.ds(k * L, L)] = zvec
        return 0

    lax.fori_loop(0, B, zb, 0)
    for j in range(RPT // B):
        pltpu.sync_copy(gbuf0, acc.at[pl.ds(s * RPT + j * B, B)])

    def zp(i, _):
        p_v[pl.ds(i * L, L)] = zvec
        return 0

    lax.fori_loop(0, RPT // L, zp, 0)
    pltpu.sync_copy(p_v.at[pl.ds(0, RPT)], rs_acc.at[pl.ds(s * RPT, RPT)])

    pltpu.sync_copy(mn_hbm, mn_v)
    pltpu.sync_copy(mx_hbm, mx_v)
    mn = mn_v[pl.ds(0, L)]
    mx = mx_v[pl.ds(0, L)]
    for i in range(1, NT):
        mn = jnp.minimum(mn, mn_v[pl.ds(i * L, L)])
        mx = jnp.maximum(mx, mx_v[pl.ds(i * L, L)])
    mn_s = jnp.min(mn)
    mx_s = jnp.max(mx)
    inv_v = 1.0 / jnp.full((L,), mx_s - mn_s, dtype=jnp.float32)

    pltpu.sync_copy(att_hbm.at[pl.ds(base, EPT)], p_v)

    def pb(i, _):
        off = i * L
        v = p_v[pl.ds(off, L)]
        p_v[pl.ds(off, L)] = jnp.exp((v - mn_s) * inv_v)
        return 0

    lax.fori_loop(0, EPT // L, pb, 0)

    plsc.subcore_barrier()

    def stage_idx(j, q):
        eoff = base + j * B
        pltpu.async_copy(row_hbm.at[pl.ds(eoff, B)], ridx.at[q], isem.at[q])
        pltpu.async_copy(col_hbm.at[pl.ds(eoff, B)], cidx.at[q], isem.at[q])

    def wait_idx(j, q):
        eoff = base + j * B
        pltpu.make_async_copy(row_hbm.at[pl.ds(eoff, B)], ridx.at[q],
                              isem.at[q]).wait()
        pltpu.make_async_copy(col_hbm.at[pl.ds(eoff, B)], cidx.at[q],
                              isem.at[q]).wait()

    def issue_gather(j, r, q):
        pltpu.async_copy(xp_hbm.at[cidx.at[q]], gbufs[r], gsem.at[r])

    def wait_gather(j, r, q):
        pltpu.make_async_copy(xp_hbm.at[cidx.at[q]], gbufs[r],
                              gsem.at[r]).wait()

    def issue_scatter(j, r, q):
        pltpu.async_copy(gbufs[r], acc.at[ridx.at[q]], ssem.at[r],
                         add=True)
        pltpu.async_copy(p_v.at[pl.ds(j * B, B)], rs_acc.at[ridx.at[q]],
                         rsem.at[r], add=True)

    def wait_scatter(j, r, q):
        pltpu.make_async_copy(gbufs[r], acc.at[ridx.at[q]],
                              ssem.at[r]).wait()
        pltpu.make_async_copy(p_v.at[pl.ds(j * B, B)], rs_acc.at[ridx.at[q]],
                              rsem.at[r]).wait()

    def scale(j, r):
        gb = gbufs[r]

        def sb(g, _):
            pv = p_v[pl.ds(j * B + g * L, L)]
            for l in range(L):
                a = pv[l]
                row = g * L + l
                for k in range(D // L):
                    gb[row, pl.ds(k * L, L)] = gb[row, pl.ds(k * L, L)] * a
            return 0

        lax.fori_loop(0, B // L, sb, 0)

    def process(j, u, stage, prep, wait_sc):
        r = u % GB
        r2 = (r + 2) % GB
        q2 = (u + 2) % IR
        q3 = (u + 3) % IR
        qm1 = (u + IR - 1) % IR
        if stage:
            stage_idx(j + 3, q3)
        wait_gather(j, r, u)
        scale(j, r)
        issue_scatter(j, r, u)
        if prep:
            if wait_sc:
                wait_scatter(j - 1, r2, qm1)
            wait_idx(j + 2, q2)
            issue_gather(j + 2, r2, q2)

    stage_idx(0, 0)
    stage_idx(1, 1)
    stage_idx(2, 2)
    wait_idx(0, 0)
    issue_gather(0, 0, 0)
    wait_idx(1, 1)
    issue_gather(1, 1, 1)
    process(0, 0, stage=True, prep=True, wait_sc=False)
    for u in range(1, IR):
        process(u, u, stage=True, prep=True, wait_sc=True)

    def body(jj, _):
        j = jj * IR
        for u in range(IR):
            process(j + u, u, stage=True, prep=True, wait_sc=True)
        return 0

    lax.fori_loop(1, 20, body, 0)

    process(120, 0, stage=True, prep=True, wait_sc=True)
    process(121, 1, stage=True, prep=True, wait_sc=True)
    process(122, 2, stage=False, prep=True, wait_sc=True)
    process(123, 3, stage=False, prep=False, wait_sc=False)
    process(124, 4, stage=False, prep=False, wait_sc=False)

    wait_scatter(122, 2, 2)
    wait_scatter(123, 0, 3)
    wait_scatter(124, 1, 4)

    plsc.subcore_barrier()

    pltpu.sync_copy(acc.at[pl.ds(s * RPT, RPT)],
                    h_out.at[c, pl.ds(s * RPT, RPT)])
    pltpu.sync_copy(rs_acc.at[pl.ds(s * RPT, RPT)],
                    rs_out.at[c, pl.ds(s * RPT, RPT)])


def _epi_body(hp_ref, rs_ref, out_ref):
    h = hp_ref[0, :N] + hp_ref[1, :N]
    rs = rs_ref[0, :N] + rs_ref[1, :N]
    out_ref[...] = h / rs


def _epilogue(h_part, rs_part):
    return pl.pallas_call(
        _epi_body,
        out_shape=jax.ShapeDtypeStruct((N, D), jnp.float32),
    )(h_part, rs_part)


def kernel(X, edge_index, W, a0, a1):
    row = edge_index[0].astype(jnp.int32)
    col = edge_index[1].astype(jnp.int32)
    a01 = jnp.concatenate([a0.T, a1.T], axis=1)
    xp, s01 = _matmul(X, W, a01)
    att, mns, mxs = _att_kernel(s01.reshape(2 * N), row, col)
    h_part, rs_part = _spmm_kernel(att, mns, mxs, row, col, xp)
    return _epilogue(h_part, rs_part.reshape(NC, NP, 1))


# speedup vs baseline: 30.8078x; 1.0251x over previous
import functools

import jax
import jax.numpy as jnp
from jax import lax
from jax.experimental import pallas as pl
from jax.experimental.pallas import tpu as pltpu
from jax.experimental.pallas import tpu_sc as plsc

N = 10000
E = 320000
D = 128
ALPHA = 0.2

NC = 2
NS = 16
NT = NC * NS
L = 16
EPT = E // NT
B = 80
NB = EPT // B
GB = 3
IR = 6
NP = 10240
RPT = NP // NS

_mesh = plsc.VectorSubcoreMesh(core_axis_name="c", subcore_axis_name="s")


def _mm_body(x_ref, w_ref, a01_ref, xp_ref, s_ref):
    xp = jnp.dot(x_ref[...], w_ref[...], preferred_element_type=jnp.float32)
    xp_ref[...] = xp
    s_ref[...] = jnp.dot(xp, a01_ref[...], preferred_element_type=jnp.float32)


def _matmul(X, W, a01):
    return pl.pallas_call(
        _mm_body,
        out_shape=[
            jax.ShapeDtypeStruct((N, D), jnp.float32),
            jax.ShapeDtypeStruct((N, 2), jnp.float32),
        ],
    )(X, W, a01)


@functools.partial(
    pl.kernel,
    out_type=[
        jax.ShapeDtypeStruct((E,), jnp.float32),
        jax.ShapeDtypeStruct((NT * L,), jnp.float32),
        jax.ShapeDtypeStruct((NT * L,), jnp.float32),
    ],
    mesh=_mesh,
    compiler_params=pltpu.CompilerParams(needs_layout_passes=False),
    scratch_types=[
        pltpu.VMEM((2 * N,), jnp.float32),
        pltpu.VMEM((EPT,), jnp.int32),
        pltpu.VMEM((EPT,), jnp.int32),
        pltpu.VMEM((EPT,), jnp.float32),
        pltpu.VMEM((L,), jnp.float32),
        pltpu.VMEM((L,), jnp.float32),
    ],
)
def _att_kernel(s01_hbm, row_hbm, col_hbm, att_hbm, mn_hbm, mx_hbm,
                s01_v, row_v, col_v, att_v, mn_v, mx_v):
    c = lax.axis_index("c")
    s = lax.axis_index("s")
    wid = c * NS + s
    base = wid * EPT
    pltpu.sync_copy(s01_hbm, s01_v)
    pltpu.sync_copy(row_hbm.at[pl.ds(base, EPT)], row_v)
    pltpu.sync_copy(col_hbm.at[pl.ds(base, EPT)], col_v)
    def body(i, carry):
        mn, mx = carry
        off = i * L
        ridx = row_v[pl.ds(off, L)]
        cidx = col_v[pl.ds(off, L)]
        v0 = plsc.load_gather(s01_v, [2 * ridx])
        v1 = plsc.load_gather(s01_v, [2 * cidx + 1])
        att = v0 + v1
        att = jnp.where(att > 0, att, ALPHA * att)
        att_v[pl.ds(off, L)] = att
        return jnp.minimum(mn, att), jnp.maximum(mx, att)

    big = jnp.full((L,), jnp.inf, dtype=jnp.float32)
    mn, mx = lax.fori_loop(0, EPT // L, body, (big, -big))
    mn_v[...] = mn
    mx_v[...] = mx
    pltpu.sync_copy(att_v, att_hbm.at[pl.ds(base, EPT)])
    pltpu.sync_copy(mn_v, mn_hbm.at[pl.ds(wid * L, L)])
    pltpu.sync_copy(mx_v, mx_hbm.at[pl.ds(wid * L, L)])


@functools.partial(
    pl.kernel,
    out_type=[
        jax.ShapeDtypeStruct((NC, NP, D), jnp.float32),
        jax.ShapeDtypeStruct((NC, NP), jnp.float32),
    ],
    mesh=_mesh,
    compiler_params=pltpu.CompilerParams(needs_layout_passes=False),
    scratch_types=[
        pltpu.VMEM_SHARED((NP, D), jnp.float32),
        pltpu.VMEM_SHARED((NP,), jnp.float32),
        pltpu.VMEM((EPT,), jnp.float32),
        pltpu.VMEM((RPT,), jnp.float32),
        pltpu.VMEM((NT * L,), jnp.float32),
        pltpu.VMEM((NT * L,), jnp.float32),
        pltpu.VMEM((IR, B), jnp.int32),
        pltpu.VMEM((IR, B), jnp.int32),
        pltpu.VMEM((B, D), jnp.float32),
        pltpu.VMEM((B, D), jnp.float32),
        pltpu.VMEM((B, D), jnp.float32),
        pltpu.SemaphoreType.DMA((GB,)),
        pltpu.SemaphoreType.DMA((GB,)),
        pltpu.SemaphoreType.DMA((GB,)),
        pltpu.SemaphoreType.DMA((IR,)),
    ],
)
def _spmm_kernel(att_hbm, mn_hbm, mx_hbm, row_hbm, col_hbm, xp_hbm,
                 h_out, rs_out,
                 acc, rs_acc, p_v, zrs, mn_v, mx_v, ridx, cidx,
                 gbuf0, gbuf1, gbuf2,
                 gsem, ssem, rsem, isem):
    gbufs = (gbuf0, gbuf1, gbuf2)
    c = lax.axis_index("c")
    s = lax.axis_index("s")
    wid = c * NS + s
    base = wid * EPT

    zvec = jnp.zeros((L,), jnp.float32)

    def zb(i, _):
        for k in range(D // L):
            gbuf0[i, pl.ds(k * L, L)] = zvec
        return 0

    lax.fori_loop(0, B, zb, 0)

    def zp(i, _):
        zrs[pl.ds(i * L, L)] = zvec
        return 0

    lax.fori_loop(0, RPT // L, zp, 0)

    for j in range(RPT // B):
        pltpu.async_copy(gbuf0, acc.at[pl.ds(s * RPT + j * B, B)],
                         ssem.at[0])
    pltpu.async_copy(zrs, rs_acc.at[pl.ds(s * RPT, RPT)], rsem.at[0])

    pltpu.sync_copy(mn_hbm, mn_v)
    pltpu.sync_copy(mx_hbm, mx_v)
    mn = mn_v[pl.ds(0, L)]
    mx = mx_v[pl.ds(0, L)]
    for i in range(1, NT):
        mn = jnp.minimum(mn, mn_v[pl.ds(i * L, L)])
        mx = jnp.maximum(mx, mx_v[pl.ds(i * L, L)])
    mn_s = jnp.min(mn)
    mx_s = jnp.max(mx)
    inv_v = 1.0 / jnp.full((L,), mx_s - mn_s, dtype=jnp.float32)

    pltpu.sync_copy(att_hbm.at[pl.ds(base, EPT)], p_v)

    def stage_idx(j, q):
        eoff = base + j * B
        pltpu.async_copy(row_hbm.at[pl.ds(eoff, B)], ridx.at[q], isem.at[q])
        pltpu.async_copy(col_hbm.at[pl.ds(eoff, B)], cidx.at[q], isem.at[q])

    def wait_idx(j, q):
        eoff = base + j * B
        pltpu.make_async_copy(row_hbm.at[pl.ds(eoff, B)], ridx.at[q],
                              isem.at[q]).wait()
        pltpu.make_async_copy(col_hbm.at[pl.ds(eoff, B)], cidx.at[q],
                              isem.at[q]).wait()

    def issue_gather(j, r, q):
        pltpu.async_copy(xp_hbm.at[cidx.at[q]], gbufs[r], gsem.at[r])

    def wait_gather(j, r, q):
        pltpu.make_async_copy(xp_hbm.at[cidx.at[q]], gbufs[r],
                              gsem.at[r]).wait()

    def issue_scatter(j, r, q):
        pltpu.async_copy(gbufs[r], acc.at[ridx.at[q]], ssem.at[r],
                         add=True)
        pltpu.async_copy(p_v.at[pl.ds(j * B, B)], rs_acc.at[ridx.at[q]],
                         rsem.at[r], add=True)

    def wait_scatter(j, r, q):
        pltpu.make_async_copy(gbufs[r], acc.at[ridx.at[q]],
                              ssem.at[r]).wait()
        pltpu.make_async_copy(p_v.at[pl.ds(j * B, B)], rs_acc.at[ridx.at[q]],
                              rsem.at[r]).wait()

    def scale(j, r):
        gb = gbufs[r]

        def sb(g, _):
            pv = p_v[pl.ds(j * B + g * L, L)]
            for l in range(L):
                a = pv[l]
                row = g * L + l
                for k in range(D // L):
                    gb[row, pl.ds(k * L, L)] = gb[row, pl.ds(k * L, L)] * a
            return 0

        lax.fori_loop(0, B // L, sb, 0)

    def process(j, u, stage, prep, wait_sc):
        r = u % GB
        r2 = (r + 2) % GB
        q2 = (u + 2) % IR
        q3 = (u + 3) % IR
        qm1 = (u + IR - 1) % IR
        if stage:
            stage_idx(j + 3, q3)
        wait_gather(j, r, u)
        scale(j, r)
        issue_scatter(j, r, u)
        if prep:
            if wait_sc:
                wait_scatter(j - 1, r2, qm1)
            wait_idx(j + 2, q2)
            issue_gather(j + 2, r2, q2)

    stage_idx(0, 0)
    stage_idx(1, 1)
    stage_idx(2, 2)
    wait_idx(0, 0)
    issue_gather(0, 0, 0)
    wait_idx(1, 1)
    issue_gather(1, 1, 1)

    def pb(i, _):
        off = i * L
        v = p_v[pl.ds(off, L)]
        p_v[pl.ds(off, L)] = jnp.exp((v - mn_s) * inv_v)
        return 0

    lax.fori_loop(0, EPT // L, pb, 0)

    for j in range(RPT // B):
        pltpu.make_async_copy(gbuf0, acc.at[pl.ds(s * RPT + j * B, B)],
                              ssem.at[0]).wait()
    pltpu.make_async_copy(zrs, rs_acc.at[pl.ds(s * RPT, RPT)],
                          rsem.at[0]).wait()

    plsc.subcore_barrier()

    process(0, 0, stage=True, prep=True, wait_sc=False)
    for u in range(1, IR):
        process(u, u, stage=True, prep=True, wait_sc=True)

    def body(jj, _):
        j = jj * IR
        for u in range(IR):
            process(j + u, u, stage=True, prep=True, wait_sc=True)
        return 0

    lax.fori_loop(1, 20, body, 0)

    process(120, 0, stage=True, prep=True, wait_sc=True)
    process(121, 1, stage=True, prep=True, wait_sc=True)
    process(122, 2, stage=False, prep=True, wait_sc=True)
    process(123, 3, stage=False, prep=False, wait_sc=False)
    process(124, 4, stage=False, prep=False, wait_sc=False)

    wait_scatter(122, 2, 2)
    wait_scatter(123, 0, 3)
    wait_scatter(124, 1, 4)

    plsc.subcore_barrier()

    pltpu.sync_copy(acc.at[pl.ds(s * RPT, RPT)],
                    h_out.at[c, pl.ds(s * RPT, RPT)])
    pltpu.sync_copy(rs_acc.at[pl.ds(s * RPT, RPT)],
                    rs_out.at[c, pl.ds(s * RPT, RPT)])


def _epi_body(hp_ref, rs_ref, out_ref):
    h = hp_ref[0, :N] + hp_ref[1, :N]
    rs = rs_ref[0, :N] + rs_ref[1, :N]
    out_ref[...] = h / rs


def _epilogue(h_part, rs_part):
    return pl.pallas_call(
        _epi_body,
        out_shape=jax.ShapeDtypeStruct((N, D), jnp.float32),
    )(h_part, rs_part)


def kernel(X, edge_index, W, a0, a1):
    row = edge_index[0].astype(jnp.int32)
    col = edge_index[1].astype(jnp.int32)
    a01 = jnp.concatenate([a0.T, a1.T], axis=1)
    xp, s01 = _matmul(X, W, a01)
    att, mns, mxs = _att_kernel(s01.reshape(2 * N), row, col)
    h_part, rs_part = _spmm_kernel(att, mns, mxs, row, col, xp)
    return _epilogue(h_part, rs_part.reshape(NC, NP, 1))
